# baseline (speedup 1.0000x reference)
import functools

import jax
import jax.numpy as jnp
from jax import lax
from jax.experimental import pallas as pl
from jax.experimental.pallas import tpu as pltpu
from jax.experimental.pallas import tpu_sc as plsc

NC = 2
NS = 16
NW = NC * NS
CH = 128


def _sc_gather(table, sidx3d, didx3d):
    n_chunks = sidx3d.shape[0]
    _, D = table.shape
    dt = table.dtype
    E = n_chunks * CH
    mesh = plsc.VectorSubcoreMesh(core_axis_name="c", subcore_axis_name="s")

    @functools.partial(
        pl.kernel,
        out_type=(
            jax.ShapeDtypeStruct((E, D), dt),
            jax.ShapeDtypeStruct((E, D), dt),
        ),
        mesh=mesh,
        scratch_types=[
            pltpu.VMEM((1, CH), jnp.int32),
            pltpu.VMEM((CH, D), dt),
            pltpu.SemaphoreType.DMA,
        ],
        compiler_params=pltpu.CompilerParams(use_tc_tiling_on_sc=False),
    )
    def k(table_hbm, sidx_hbm, didx_hbm, outs_hbm, outd_hbm, idx_v, rows_v, sem):
        wid = lax.axis_index("s") * NC + lax.axis_index("c")
        nloc = (n_chunks - wid + NW - 1) // NW

        @pl.loop(0, nloc)
        def _(j):
            c = wid + j * NW
            pltpu.sync_copy(sidx_hbm.at[c], idx_v)
            pltpu.async_copy(table_hbm.at[idx_v.at[0]], rows_v, sem).wait()
            pltpu.sync_copy(rows_v, outs_hbm.at[pl.ds(c * CH, CH)])
            pltpu.sync_copy(didx_hbm.at[c], idx_v)
            pltpu.async_copy(table_hbm.at[idx_v.at[0]], rows_v, sem).wait()
            pltpu.sync_copy(rows_v, outd_hbm.at[pl.ds(c * CH, CH)])

    return k(table, sidx3d, didx3d)


def _sc_scatter(ef, didx3d, zeros, n_nodes):
    n_chunks = didx3d.shape[0]
    D = ef.shape[1]
    rpt = (n_nodes // NS) & ~7
    tail = n_nodes - rpt * NS
    mesh = plsc.VectorSubcoreMesh(core_axis_name="c", subcore_axis_name="s")

    @functools.partial(
        pl.kernel,
        out_type=jax.ShapeDtypeStruct((NC * n_nodes, D), jnp.float32),
        mesh=mesh,
        scratch_types=[
            pltpu.VMEM((1, CH), jnp.int32),
            pltpu.VMEM((CH, D), jnp.float32),
            pltpu.VMEM_SHARED((n_nodes, D), jnp.float32),
        ],
    )
    def k(ef_hbm, didx_hbm, zeros_hbm, out_hbm, idx_v, rows_v, acc_s):
        cid = lax.axis_index("c")
        sid = lax.axis_index("s")
        wid = sid * NC + cid
        pltpu.sync_copy(zeros_hbm.at[pl.ds(sid * rpt, rpt)],
                        acc_s.at[pl.ds(sid * rpt, rpt)])

        @pl.when(jnp.logical_and(sid == NS - 1, tail > 0))
        def _():
            pltpu.sync_copy(zeros_hbm.at[pl.ds(NS * rpt, tail)],
                            acc_s.at[pl.ds(NS * rpt, tail)])

        plsc.subcore_barrier()
        nloc = (n_chunks - wid + NW - 1) // NW

        @pl.loop(0, nloc)
        def _(j):
            c = wid + j * NW
            pltpu.sync_copy(didx_hbm.at[c], idx_v)
            pltpu.sync_copy(ef_hbm.at[pl.ds(c * CH, CH)], rows_v)
            pltpu.sync_copy(rows_v, acc_s.at[idx_v.at[0]], add=True)

        plsc.subcore_barrier()
        pltpu.sync_copy(acc_s.at[pl.ds(sid * rpt, rpt)],
                        out_hbm.at[pl.ds(cid * n_nodes + sid * rpt, rpt)])

        @pl.when(jnp.logical_and(sid == NS - 1, tail > 0))
        def _():
            pltpu.sync_copy(acc_s.at[pl.ds(NS * rpt, tail)],
                            out_hbm.at[pl.ds(cid * n_nodes + NS * rpt, tail)])

    return k(ef, didx3d, zeros)


def _layer_norm(h, g, beta):
    mu = jnp.mean(h, axis=-1, keepdims=True)
    var = jnp.mean((h - mu) * (h - mu), axis=-1, keepdims=True)
    return (h - mu) * lax.rsqrt(var + 1e-5) * g + beta


def _tc_edge_mlp(src, dst, ef, W0s, W0d, W0e, b0, W1, b1, W2, b2, g, beta):
    E, D = ef.shape
    BE = 2000
    grid = (E // BE,)

    def body(src_r, dst_r, ef_r, w0s_r, w0d_r, w0e_r, b0_r, w1_r, b1_r,
             w2_r, b2_r, g_r, beta_r, out_r):
        dot = functools.partial(jnp.dot, preferred_element_type=jnp.float32)

        def unpack(xi):
            lo = lax.bitcast_convert_type(xi << 16, jnp.float32)
            hi = lax.bitcast_convert_type(xi & jnp.int32(-65536), jnp.float32)
            return jnp.concatenate([lo, hi], axis=-1).astype(jnp.bfloat16)

        src = unpack(src_r[...])
        dst = unpack(dst_r[...])
        x = (dot(src, w0s_r[...]) + dot(dst, w0d_r[...])
             + dot(ef_r[...], w0e_r[...]) + b0_r[...])
        h = jnp.maximum(x, 0.0)
        h = jnp.maximum(dot(h, w1_r[...]) + b1_r[...], 0.0)
        h = dot(h, w2_r[...]) + b2_r[...]
        out_r[...] = ef_r[...] + _layer_norm(h, g_r[...], beta_r[...])

    blk = lambda i: (i, 0)
    full = lambda i: (0, 0)
    return pl.pallas_call(
        body,
        grid=grid,
        in_specs=[
            pl.BlockSpec((BE, D // 2), blk),
            pl.BlockSpec((BE, D // 2), blk),
            pl.BlockSpec((BE, D), blk),
            pl.BlockSpec((D, D), full),
            pl.BlockSpec((D, D), full),
            pl.BlockSpec((D, D), full),
            pl.BlockSpec((1, D), full),
            pl.BlockSpec((D, D), full),
            pl.BlockSpec((1, D), full),
            pl.BlockSpec((D, D), full),
            pl.BlockSpec((1, D), full),
            pl.BlockSpec((1, D), full),
            pl.BlockSpec((1, D), full),
        ],
        out_specs=pl.BlockSpec((BE, D), blk),
        out_shape=jax.ShapeDtypeStruct((E, D), jnp.float32),
    )(src, dst, ef, W0s, W0d, W0e, b0, W1, b1, W2, b2, g, beta)


def _tc_node_mlp(nf, parts, W0n, W0a, b0, W1, b1, W2, b2, g, beta):
    N, D = nf.shape
    BN = 1000
    grid = (N // BN,)

    def body(nf_r, p0_r, p1_r, w0n_r, w0a_r, b0_r, w1_r, b1_r,
             w2_r, b2_r, g_r, beta_r, out_r):
        dot = functools.partial(jnp.dot, preferred_element_type=jnp.float32)
        agg = p0_r[...] + p1_r[...]
        x = dot(nf_r[...], w0n_r[...]) + dot(agg, w0a_r[...]) + b0_r[...]
        h = jnp.maximum(x, 0.0)
        h = jnp.maximum(dot(h, w1_r[...]) + b1_r[...], 0.0)
        h = dot(h, w2_r[...]) + b2_r[...]
        out_r[...] = nf_r[...] + _layer_norm(h, g_r[...], beta_r[...])

    blk = lambda i: (i, 0)
    full = lambda i: (0, 0)
    p0 = lax.slice_in_dim(parts, 0, N, axis=0)
    p1 = lax.slice_in_dim(parts, N, 2 * N, axis=0)
    return pl.pallas_call(
        body,
        grid=grid,
        in_specs=[
            pl.BlockSpec((BN, D), blk),
            pl.BlockSpec((BN, D), blk),
            pl.BlockSpec((BN, D), blk),
            pl.BlockSpec((D, D), full),
            pl.BlockSpec((D, D), full),
            pl.BlockSpec((1, D), full),
            pl.BlockSpec((D, D), full),
            pl.BlockSpec((1, D), full),
            pl.BlockSpec((D, D), full),
            pl.BlockSpec((1, D), full),
            pl.BlockSpec((1, D), full),
            pl.BlockSpec((1, D), full),
        ],
        out_specs=pl.BlockSpec((BN, D), blk),
        out_shape=jax.ShapeDtypeStruct((N, D), jnp.float32),
    )(nf, p0, p1, W0n, W0a, b0, W1, b1, W2, b2, g, beta)


def kernel(node_features, edge_features, edge_index,
           edge_W0, edge_b0, edge_W1, edge_b1, edge_W2, edge_b2,
           edge_g, edge_beta,
           node_W0, node_b0, node_W1, node_b1, node_W2, node_b2,
           node_g, node_beta):
    N, DN = node_features.shape
    E, DE = edge_features.shape
    P = edge_W0.shape[0]
    n_chunks = E // CH
    sidx3d = edge_index[0].reshape(n_chunks, 1, CH)
    didx3d = edge_index[1].reshape(n_chunks, 1, CH)
    zeros = jnp.zeros((N, DE), dtype=jnp.float32)
    row = lambda b: b.reshape(1, -1)

    bf16 = jnp.bfloat16
    nf = node_features
    ef = edge_features
    def pack_cols(x16):
        h = x16.shape[1] // 2
        u = lax.bitcast_convert_type(x16, jnp.uint16).astype(jnp.uint32)
        packed = u[:, :h] | (u[:, h:] << 16)
        return lax.bitcast_convert_type(packed, jnp.int32)

    for i in range(P):
        src_rows, dst_rows = _sc_gather(pack_cols(nf.astype(bf16)),
                                        sidx3d, didx3d)
        ef = _tc_edge_mlp(
            src_rows, dst_rows, ef,
            edge_W0[i, :DN].astype(bf16), edge_W0[i, DN:2 * DN].astype(bf16),
            edge_W0[i, 2 * DN:],
            row(edge_b0[i]), edge_W1[i], row(edge_b1[i]),
            edge_W2[i], row(edge_b2[i]), row(edge_g[i]), row(edge_beta[i]))
        parts = _sc_scatter(ef, didx3d, zeros, N)
        nf = _tc_node_mlp(
            nf, parts,
            node_W0[i, :DN], node_W0[i, DN:],
            row(node_b0[i]), node_W1[i], row(node_b1[i]),
            node_W2[i], row(node_b2[i]), row(node_g[i]), row(node_beta[i]))
    return nf


# speedup vs baseline: 1.2833x; 1.2833x over previous
import functools

import jax
import jax.numpy as jnp
from jax import lax
from jax.experimental import pallas as pl
from jax.experimental.pallas import tpu as pltpu
from jax.experimental.pallas import tpu_sc as plsc

NC = 2
NS = 16
NW = NC * NS
CH = 128
PIECES = 4


def _sc_gather(table, sidx3d, didx3d):
    n_chunks = sidx3d.shape[0]
    _, D = table.shape
    Ep = n_chunks * CH
    mesh = plsc.VectorSubcoreMesh(core_axis_name="c", subcore_axis_name="s")

    @functools.partial(
        pl.kernel,
        out_type=(
            jax.ShapeDtypeStruct((Ep, D), jnp.float32),
            jax.ShapeDtypeStruct((Ep, D), jnp.float32),
        ),
        mesh=mesh,
        scratch_types=[
            pltpu.VMEM((1, CH), jnp.int32),
            pltpu.VMEM((CH, D), jnp.float32),
            pltpu.SemaphoreType.DMA,
        ],
    )
    def k(table_hbm, sidx_hbm, didx_hbm, outs_hbm, outd_hbm, idx_v, rows_v, sem):
        wid = lax.axis_index("s") * NC + lax.axis_index("c")
        nloc = (n_chunks - wid + NW - 1) // NW

        @pl.loop(0, nloc)
        def _(j):
            c = wid + j * NW
            pltpu.sync_copy(sidx_hbm.at[c], idx_v)
            pltpu.async_copy(table_hbm.at[idx_v.at[0]], rows_v, sem).wait()
            pltpu.sync_copy(rows_v, outs_hbm.at[pl.ds(c * CH, CH)])
            pltpu.sync_copy(didx_hbm.at[c], idx_v)
            pltpu.async_copy(table_hbm.at[idx_v.at[0]], rows_v, sem).wait()
            pltpu.sync_copy(rows_v, outd_hbm.at[pl.ds(c * CH, CH)])

    return k(table, sidx3d, didx3d)


def _sc_scatter(ef_piece, didx3d, zeros, n_nodes):
    n_chunks = didx3d.shape[0]
    D = ef_piece.shape[1]
    rpt = (n_nodes // NS) & ~7
    tail = n_nodes - rpt * NS
    mesh = plsc.VectorSubcoreMesh(core_axis_name="c", subcore_axis_name="s")

    @functools.partial(
        pl.kernel,
        out_type=jax.ShapeDtypeStruct((NC * n_nodes, D), jnp.float32),
        mesh=mesh,
        scratch_types=[
            pltpu.VMEM((1, CH), jnp.int32),
            pltpu.VMEM((CH, D), jnp.float32),
            pltpu.VMEM_SHARED((n_nodes, D), jnp.float32),
        ],
    )
    def k(ef_hbm, didx_hbm, zeros_hbm, out_hbm, idx_v, rows_v, acc_s):
        cid = lax.axis_index("c")
        sid = lax.axis_index("s")
        wid = sid * NC + cid
        pltpu.sync_copy(zeros_hbm.at[pl.ds(sid * rpt, rpt)],
                        acc_s.at[pl.ds(sid * rpt, rpt)])

        @pl.when(jnp.logical_and(sid == NS - 1, tail > 0))
        def _():
            pltpu.sync_copy(zeros_hbm.at[pl.ds(NS * rpt, tail)],
                            acc_s.at[pl.ds(NS * rpt, tail)])

        plsc.subcore_barrier()
        nloc = (n_chunks - wid + NW - 1) // NW

        @pl.loop(0, nloc)
        def _(j):
            c = wid + j * NW
            pltpu.sync_copy(didx_hbm.at[c], idx_v)
            pltpu.sync_copy(ef_hbm.at[pl.ds(c * CH, CH)], rows_v)
            pltpu.sync_copy(rows_v, acc_s.at[idx_v.at[0]], add=True)

        plsc.subcore_barrier()
        pltpu.sync_copy(acc_s.at[pl.ds(sid * rpt, rpt)],
                        out_hbm.at[pl.ds(cid * n_nodes + sid * rpt, rpt)])

        @pl.when(jnp.logical_and(sid == NS - 1, tail > 0))
        def _():
            pltpu.sync_copy(acc_s.at[pl.ds(NS * rpt, tail)],
                            out_hbm.at[pl.ds(cid * n_nodes + NS * rpt, tail)])

    return k(ef_piece, didx3d, zeros)


def _layer_norm(h, g, beta):
    mu = jnp.mean(h, axis=-1, keepdims=True)
    var = jnp.mean((h - mu) * (h - mu), axis=-1, keepdims=True)
    return (h - mu) * lax.rsqrt(var + 1e-5) * g + beta


def _tc_edge_mlp(src, dst, ef, W0s, W0d, W0e, b0, W1, b1, W2, b2, g, beta):
    Ep, D = ef.shape
    BE = 2000
    grid = (Ep // BE,)

    def body(src_r, dst_r, ef_r, w0s_r, w0d_r, w0e_r, b0_r, w1_r, b1_r,
             w2_r, b2_r, g_r, beta_r, out_r):
        dot = functools.partial(jnp.dot, preferred_element_type=jnp.float32)
        x = (dot(src_r[...], w0s_r[...]) + dot(dst_r[...], w0d_r[...])
             + dot(ef_r[...], w0e_r[...]) + b0_r[...])
        h = jnp.maximum(x, 0.0)
        h = jnp.maximum(dot(h, w1_r[...]) + b1_r[...], 0.0)
        h = dot(h, w2_r[...]) + b2_r[...]
        out_r[...] = ef_r[...] + _layer_norm(h, g_r[...], beta_r[...])

    blk = lambda i: (i, 0)
    full = lambda i: (0, 0)
    return pl.pallas_call(
        body,
        grid=grid,
        in_specs=[
            pl.BlockSpec((BE, D), blk),
            pl.BlockSpec((BE, D), blk),
            pl.BlockSpec((BE, D), blk),
            pl.BlockSpec((D, D), full),
            pl.BlockSpec((D, D), full),
            pl.BlockSpec((D, D), full),
            pl.BlockSpec((1, D), full),
            pl.BlockSpec((D, D), full),
            pl.BlockSpec((1, D), full),
            pl.BlockSpec((D, D), full),
            pl.BlockSpec((1, D), full),
            pl.BlockSpec((1, D), full),
            pl.BlockSpec((1, D), full),
        ],
        out_specs=pl.BlockSpec((BE, D), blk),
        out_shape=jax.ShapeDtypeStruct((Ep, D), jnp.float32),
    )(src, dst, ef, W0s, W0d, W0e, b0, W1, b1, W2, b2, g, beta)


def _tc_node_mlp(nf, parts_list, W0n, W0a, b0, W1, b1, W2, b2, g, beta):
    N, D = nf.shape
    BN = 1000
    grid = (N // BN,)
    n_parts = 2 * len(parts_list)

    def body(*refs):
        nf_r = refs[0]
        part_rs = refs[1:1 + n_parts]
        (w0n_r, w0a_r, b0_r, w1_r, b1_r, w2_r, b2_r, g_r, beta_r,
         out_r) = refs[1 + n_parts:]
        dot = functools.partial(jnp.dot, preferred_element_type=jnp.float32)
        agg = part_rs[0][...]
        for pr in part_rs[1:]:
            agg = agg + pr[...]
        x = dot(nf_r[...], w0n_r[...]) + dot(agg, w0a_r[...]) + b0_r[...]
        h = jnp.maximum(x, 0.0)
        h = jnp.maximum(dot(h, w1_r[...]) + b1_r[...], 0.0)
        h = dot(h, w2_r[...]) + b2_r[...]
        out_r[...] = nf_r[...] + _layer_norm(h, g_r[...], beta_r[...])

    blk = lambda i: (i, 0)
    full = lambda i: (0, 0)
    flat_parts = []
    for parts in parts_list:
        flat_parts.append(lax.slice_in_dim(parts, 0, N, axis=0))
        flat_parts.append(lax.slice_in_dim(parts, N, 2 * N, axis=0))
    return pl.pallas_call(
        body,
        grid=grid,
        in_specs=(
            [pl.BlockSpec((BN, D), blk)] * (1 + n_parts)
            + [
                pl.BlockSpec((D, D), full),
                pl.BlockSpec((D, D), full),
                pl.BlockSpec((1, D), full),
                pl.BlockSpec((D, D), full),
                pl.BlockSpec((1, D), full),
                pl.BlockSpec((D, D), full),
                pl.BlockSpec((1, D), full),
                pl.BlockSpec((1, D), full),
                pl.BlockSpec((1, D), full),
            ]
        ),
        out_specs=pl.BlockSpec((BN, D), blk),
        out_shape=jax.ShapeDtypeStruct((N, D), jnp.float32),
    )(nf, *flat_parts, W0n, W0a, b0, W1, b1, W2, b2, g, beta)


def kernel(node_features, edge_features, edge_index,
           edge_W0, edge_b0, edge_W1, edge_b1, edge_W2, edge_b2,
           edge_g, edge_beta,
           node_W0, node_b0, node_W1, node_b1, node_W2, node_b2,
           node_g, node_beta):
    N, DN = node_features.shape
    E, DE = edge_features.shape
    P = edge_W0.shape[0]
    n_chunks = E // CH
    cpp = n_chunks // PIECES
    sidx3d = edge_index[0].reshape(n_chunks, 1, CH)
    didx3d = edge_index[1].reshape(n_chunks, 1, CH)
    sidx_p = [sidx3d[k * cpp:(k + 1) * cpp] for k in range(PIECES)]
    didx_p = [didx3d[k * cpp:(k + 1) * cpp] for k in range(PIECES)]
    zeros = jnp.zeros((N, DE), dtype=jnp.float32)
    row = lambda b: b.reshape(1, -1)
    Ep = cpp * CH

    nf = node_features
    ef_p = [lax.slice_in_dim(edge_features, k * Ep, (k + 1) * Ep, axis=0)
            for k in range(PIECES)]
    for i in range(P):
        ew = (edge_W0[i, :DN], edge_W0[i, DN:2 * DN], edge_W0[i, 2 * DN:],
              row(edge_b0[i]), edge_W1[i], row(edge_b1[i]),
              edge_W2[i], row(edge_b2[i]), row(edge_g[i]), row(edge_beta[i]))
        parts_list = []
        new_ef_p = []
        for k in range(PIECES):
            src_rows, dst_rows = _sc_gather(nf, sidx_p[k], didx_p[k])
            efk = _tc_edge_mlp(src_rows, dst_rows, ef_p[k], *ew)
            new_ef_p.append(efk)
            parts_list.append(_sc_scatter(efk, didx_p[k], zeros, N))
        ef_p = new_ef_p
        nf = _tc_node_mlp(
            nf, parts_list,
            node_W0[i, :DN], node_W0[i, DN:],
            row(node_b0[i]), node_W1[i], row(node_b1[i]),
            node_W2[i], row(node_b2[i]), row(node_g[i]), row(node_beta[i]))
    return nf


# speedup vs baseline: 1.3509x; 1.0527x over previous
import functools

import jax
import jax.numpy as jnp
from jax import lax
from jax.experimental import pallas as pl
from jax.experimental.pallas import tpu as pltpu
from jax.experimental.pallas import tpu_sc as plsc

NC = 2
NS = 16
NW = NC * NS
CH = 128
PIECES = 2


def _sc_gather(table, sidx3d, didx3d):
    n_chunks = sidx3d.shape[0]
    _, D = table.shape
    Ep = n_chunks * CH
    mesh = plsc.VectorSubcoreMesh(core_axis_name="c", subcore_axis_name="s")

    @functools.partial(
        pl.kernel,
        out_type=(
            jax.ShapeDtypeStruct((Ep, D), jnp.float32),
            jax.ShapeDtypeStruct((Ep, D), jnp.float32),
        ),
        mesh=mesh,
        scratch_types=[
            pltpu.VMEM((1, CH), jnp.int32),
            pltpu.VMEM((CH, D), jnp.float32),
            pltpu.SemaphoreType.DMA,
        ],
    )
    def k(table_hbm, sidx_hbm, didx_hbm, outs_hbm, outd_hbm, idx_v, rows_v, sem):
        wid = lax.axis_index("s") * NC + lax.axis_index("c")
        nloc = (n_chunks - wid + NW - 1) // NW

        @pl.loop(0, nloc)
        def _(j):
            c = wid + j * NW
            pltpu.sync_copy(sidx_hbm.at[c], idx_v)
            pltpu.async_copy(table_hbm.at[idx_v.at[0]], rows_v, sem).wait()
            pltpu.sync_copy(rows_v, outs_hbm.at[pl.ds(c * CH, CH)])
            pltpu.sync_copy(didx_hbm.at[c], idx_v)
            pltpu.async_copy(table_hbm.at[idx_v.at[0]], rows_v, sem).wait()
            pltpu.sync_copy(rows_v, outd_hbm.at[pl.ds(c * CH, CH)])

    return k(table, sidx3d, didx3d)


def _sc_scatter(ef_piece, didx3d, zeros, n_nodes):
    n_chunks = didx3d.shape[0]
    D = ef_piece.shape[1]
    rpt = (n_nodes // NS) & ~7
    tail = n_nodes - rpt * NS
    mesh = plsc.VectorSubcoreMesh(core_axis_name="c", subcore_axis_name="s")

    @functools.partial(
        pl.kernel,
        out_type=jax.ShapeDtypeStruct((NC * n_nodes, D), jnp.float32),
        mesh=mesh,
        scratch_types=[
            pltpu.VMEM((1, CH), jnp.int32),
            pltpu.VMEM((CH, D), jnp.float32),
            pltpu.VMEM_SHARED((n_nodes, D), jnp.float32),
        ],
    )
    def k(ef_hbm, didx_hbm, zeros_hbm, out_hbm, idx_v, rows_v, acc_s):
        cid = lax.axis_index("c")
        sid = lax.axis_index("s")
        wid = sid * NC + cid
        pltpu.sync_copy(zeros_hbm.at[pl.ds(sid * rpt, rpt)],
                        acc_s.at[pl.ds(sid * rpt, rpt)])

        @pl.when(jnp.logical_and(sid == NS - 1, tail > 0))
        def _():
            pltpu.sync_copy(zeros_hbm.at[pl.ds(NS * rpt, tail)],
                            acc_s.at[pl.ds(NS * rpt, tail)])

        plsc.subcore_barrier()
        nloc = (n_chunks - wid + NW - 1) // NW

        @pl.loop(0, nloc)
        def _(j):
            c = wid + j * NW
            pltpu.sync_copy(didx_hbm.at[c], idx_v)
            pltpu.sync_copy(ef_hbm.at[pl.ds(c * CH, CH)], rows_v)
            pltpu.sync_copy(rows_v, acc_s.at[idx_v.at[0]], add=True)

        plsc.subcore_barrier()
        pltpu.sync_copy(acc_s.at[pl.ds(sid * rpt, rpt)],
                        out_hbm.at[pl.ds(cid * n_nodes + sid * rpt, rpt)])

        @pl.when(jnp.logical_and(sid == NS - 1, tail > 0))
        def _():
            pltpu.sync_copy(acc_s.at[pl.ds(NS * rpt, tail)],
                            out_hbm.at[pl.ds(cid * n_nodes + NS * rpt, tail)])

    return k(ef_piece, didx3d, zeros)


def _layer_norm(h, g, beta):
    mu = jnp.mean(h, axis=-1, keepdims=True)
    var = jnp.mean((h - mu) * (h - mu), axis=-1, keepdims=True)
    return (h - mu) * lax.rsqrt(var + 1e-5) * g + beta


def _tc_edge_mlp(src, dst, ef, W0s, W0d, W0e, b0, W1, b1, W2, b2, g, beta):
    Ep, D = ef.shape
    BE = 2000
    grid = (Ep // BE,)

    def body(src_r, dst_r, ef_r, w0s_r, w0d_r, w0e_r, b0_r, w1_r, b1_r,
             w2_r, b2_r, g_r, beta_r, out_r):
        dot = functools.partial(jnp.dot, preferred_element_type=jnp.float32)
        x = (dot(src_r[...], w0s_r[...]) + dot(dst_r[...], w0d_r[...])
             + dot(ef_r[...], w0e_r[...]) + b0_r[...])
        h = jnp.maximum(x, 0.0)
        h = jnp.maximum(dot(h, w1_r[...]) + b1_r[...], 0.0)
        h = dot(h, w2_r[...]) + b2_r[...]
        out_r[...] = ef_r[...] + _layer_norm(h, g_r[...], beta_r[...])

    blk = lambda i: (i, 0)
    full = lambda i: (0, 0)
    return pl.pallas_call(
        body,
        grid=grid,
        in_specs=[
            pl.BlockSpec((BE, D), blk),
            pl.BlockSpec((BE, D), blk),
            pl.BlockSpec((BE, D), blk),
            pl.BlockSpec((D, D), full),
            pl.BlockSpec((D, D), full),
            pl.BlockSpec((D, D), full),
            pl.BlockSpec((1, D), full),
            pl.BlockSpec((D, D), full),
            pl.BlockSpec((1, D), full),
            pl.BlockSpec((D, D), full),
            pl.BlockSpec((1, D), full),
            pl.BlockSpec((1, D), full),
            pl.BlockSpec((1, D), full),
        ],
        out_specs=pl.BlockSpec((BE, D), blk),
        out_shape=jax.ShapeDtypeStruct((Ep, D), jnp.float32),
    )(src, dst, ef, W0s, W0d, W0e, b0, W1, b1, W2, b2, g, beta)


def _tc_node_mlp(nf, parts_list, W0n, W0a, b0, W1, b1, W2, b2, g, beta):
    N, D = nf.shape
    BN = 1000
    grid = (N // BN,)
    n_parts = 2 * len(parts_list)

    def body(*refs):
        nf_r = refs[0]
        part_rs = refs[1:1 + n_parts]
        (w0n_r, w0a_r, b0_r, w1_r, b1_r, w2_r, b2_r, g_r, beta_r,
         out_r) = refs[1 + n_parts:]
        dot = functools.partial(jnp.dot, preferred_element_type=jnp.float32)
        agg = part_rs[0][...]
        for pr in part_rs[1:]:
            agg = agg + pr[...]
        x = dot(nf_r[...], w0n_r[...]) + dot(agg, w0a_r[...]) + b0_r[...]
        h = jnp.maximum(x, 0.0)
        h = jnp.maximum(dot(h, w1_r[...]) + b1_r[...], 0.0)
        h = dot(h, w2_r[...]) + b2_r[...]
        out_r[...] = nf_r[...] + _layer_norm(h, g_r[...], beta_r[...])

    blk = lambda i: (i, 0)
    full = lambda i: (0, 0)
    flat_parts = []
    for parts in parts_list:
        flat_parts.append(lax.slice_in_dim(parts, 0, N, axis=0))
        flat_parts.append(lax.slice_in_dim(parts, N, 2 * N, axis=0))
    return pl.pallas_call(
        body,
        grid=grid,
        in_specs=(
            [pl.BlockSpec((BN, D), blk)] * (1 + n_parts)
            + [
                pl.BlockSpec((D, D), full),
                pl.BlockSpec((D, D), full),
                pl.BlockSpec((1, D), full),
                pl.BlockSpec((D, D), full),
                pl.BlockSpec((1, D), full),
                pl.BlockSpec((D, D), full),
                pl.BlockSpec((1, D), full),
                pl.BlockSpec((1, D), full),
                pl.BlockSpec((1, D), full),
            ]
        ),
        out_specs=pl.BlockSpec((BN, D), blk),
        out_shape=jax.ShapeDtypeStruct((N, D), jnp.float32),
    )(nf, *flat_parts, W0n, W0a, b0, W1, b1, W2, b2, g, beta)


def kernel(node_features, edge_features, edge_index,
           edge_W0, edge_b0, edge_W1, edge_b1, edge_W2, edge_b2,
           edge_g, edge_beta,
           node_W0, node_b0, node_W1, node_b1, node_W2, node_b2,
           node_g, node_beta):
    N, DN = node_features.shape
    E, DE = edge_features.shape
    P = edge_W0.shape[0]
    n_chunks = E // CH
    cpp = n_chunks // PIECES
    sidx3d = edge_index[0].reshape(n_chunks, 1, CH)
    didx3d = edge_index[1].reshape(n_chunks, 1, CH)
    sidx_p = [sidx3d[k * cpp:(k + 1) * cpp] for k in range(PIECES)]
    didx_p = [didx3d[k * cpp:(k + 1) * cpp] for k in range(PIECES)]
    zeros = jnp.zeros((N, DE), dtype=jnp.float32)
    row = lambda b: b.reshape(1, -1)
    Ep = cpp * CH

    nf = node_features
    ef_p = [lax.slice_in_dim(edge_features, k * Ep, (k + 1) * Ep, axis=0)
            for k in range(PIECES)]
    for i in range(P):
        ew = (edge_W0[i, :DN], edge_W0[i, DN:2 * DN], edge_W0[i, 2 * DN:],
              row(edge_b0[i]), edge_W1[i], row(edge_b1[i]),
              edge_W2[i], row(edge_b2[i]), row(edge_g[i]), row(edge_beta[i]))
        parts_list = []
        new_ef_p = []
        for k in range(PIECES):
            src_rows, dst_rows = _sc_gather(nf, sidx_p[k], didx_p[k])
            efk = _tc_edge_mlp(src_rows, dst_rows, ef_p[k], *ew)
            new_ef_p.append(efk)
            parts_list.append(_sc_scatter(efk, didx_p[k], zeros, N))
        ef_p = new_ef_p
        nf = _tc_node_mlp(
            nf, parts_list,
            node_W0[i, :DN], node_W0[i, DN:],
            row(node_b0[i]), node_W1[i], row(node_b1[i]),
            node_W2[i], row(node_b2[i]), row(node_g[i]), row(node_beta[i]))
    return nf
